# Initial kernel scaffold; baseline (speedup 1.0000x reference)
import jax
import jax.numpy as jnp
from jax.experimental import pallas as pl


def kernel(x, centroid_emb, active_mask):
    raise NotImplementedError("write your pallas kernel here")


# speedup vs baseline: 1.0816x; 1.0816x over previous
import functools

import jax
import jax.numpy as jnp
from jax.experimental import pallas as pl

B, P, D = 4096, 8192, 64
BLOCK_B = 256


def _centroid_kernel(x_ref, c_ref, bias_ref, out_ref):
    x = x_ref[...]
    c = c_ref[...]
    bias = bias_ref[...]

    xn = x / (jnp.sqrt(jnp.sum(x * x, axis=-1, keepdims=True)) + 1e-12)
    cn = c / (jnp.sqrt(jnp.sum(c * c, axis=-1, keepdims=True)) + 1e-12)

    sim = jax.lax.dot_general(
        xn, cn, (((1,), (1,)), ((), ())),
        preferred_element_type=jnp.float32)
    sim = sim + bias

    m = jnp.max(sim, axis=-1, keepdims=True)
    e = jnp.exp(sim - m)
    s = jnp.sum(e, axis=-1, keepdims=True)
    attn = e / s

    out_ref[...] = jax.lax.dot_general(
        attn, c, (((1,), (0,)), ((), ())),
        preferred_element_type=jnp.float32)


@jax.jit
def kernel(x, centroid_emb, active_mask):
    bias = jnp.where(active_mask, 0.0, -1e9).astype(jnp.float32).reshape(1, P)
    return pl.pallas_call(
        _centroid_kernel,
        grid=(B // BLOCK_B,),
        in_specs=[
            pl.BlockSpec((BLOCK_B, D), lambda i: (i, 0)),
            pl.BlockSpec((P, D), lambda i: (0, 0)),
            pl.BlockSpec((1, P), lambda i: (0, 0)),
        ],
        out_specs=pl.BlockSpec((BLOCK_B, D), lambda i: (i, 0)),
        out_shape=jax.ShapeDtypeStruct((B, D), jnp.float32),
    )(x, centroid_emb, bias)


# speedup vs baseline: 1.8431x; 1.7041x over previous
import functools

import jax
import jax.numpy as jnp
from jax.experimental import pallas as pl
from jax.experimental.pallas import tpu as pltpu

B, P, D = 4096, 8192, 64
BLOCK_B = 256


def _centroid_kernel(x_ref, c_ref, bias_ref, out_ref):
    x = x_ref[...]
    c = c_ref[...]
    bias = bias_ref[...]

    xn = x / (jnp.sqrt(jnp.sum(x * x, axis=-1, keepdims=True)) + 1e-12)
    cn = c / (jnp.sqrt(jnp.sum(c * c, axis=-1, keepdims=True)) + 1e-12)

    sim = jax.lax.dot_general(
        xn, cn, (((1,), (1,)), ((), ())),
        preferred_element_type=jnp.float32)
    e = jnp.exp(sim + bias)
    s = jnp.sum(e, axis=-1, keepdims=True)

    ctx = jax.lax.dot_general(
        e, c, (((1,), (0,)), ((), ())),
        preferred_element_type=jnp.float32)
    out_ref[...] = ctx / s


@jax.jit
def kernel(x, centroid_emb, active_mask):
    bias = jnp.where(active_mask, 0.0, -1e9).astype(jnp.float32).reshape(1, P)
    return pl.pallas_call(
        _centroid_kernel,
        grid=(B // BLOCK_B,),
        in_specs=[
            pl.BlockSpec((BLOCK_B, D), lambda i: (i, 0)),
            pl.BlockSpec((P, D), lambda i: (0, 0)),
            pl.BlockSpec((1, P), lambda i: (0, 0)),
        ],
        out_specs=pl.BlockSpec((BLOCK_B, D), lambda i: (i, 0)),
        out_shape=jax.ShapeDtypeStruct((B, D), jnp.float32),
        compiler_params=pltpu.CompilerParams(
            dimension_semantics=("parallel",)),
    )(x, centroid_emb, bias)


# speedup vs baseline: 1.9484x; 1.0571x over previous
import jax
import jax.numpy as jnp
from jax.experimental import pallas as pl
from jax.experimental.pallas import tpu as pltpu

B, P, D = 4096, 8192, 64
BLOCK_B = 256


def _normalize_kernel(c_ref, cn_ref):
    c = c_ref[...]
    cn_ref[...] = c / (jnp.sqrt(jnp.sum(c * c, axis=-1, keepdims=True)) + 1e-12)


def _centroid_kernel(x_ref, cn_ref, c_ref, bias_ref, out_ref):
    x = x_ref[...]
    cn = cn_ref[...]
    c = c_ref[...]
    bias = bias_ref[...]

    xn = x / (jnp.sqrt(jnp.sum(x * x, axis=-1, keepdims=True)) + 1e-12)

    sim = jax.lax.dot_general(
        xn, cn, (((1,), (1,)), ((), ())),
        preferred_element_type=jnp.float32)
    e = jnp.exp(sim + bias)
    s = jnp.sum(e, axis=-1, keepdims=True)

    ctx = jax.lax.dot_general(
        e, c, (((1,), (0,)), ((), ())),
        preferred_element_type=jnp.float32)
    out_ref[...] = ctx / s


@jax.jit
def kernel(x, centroid_emb, active_mask):
    bias = jnp.where(active_mask, 0.0, -1e9).astype(jnp.float32).reshape(1, P)
    cn = pl.pallas_call(
        _normalize_kernel,
        out_shape=jax.ShapeDtypeStruct((P, D), jnp.float32),
    )(centroid_emb)
    return pl.pallas_call(
        _centroid_kernel,
        grid=(B // BLOCK_B,),
        in_specs=[
            pl.BlockSpec((BLOCK_B, D), lambda i: (i, 0)),
            pl.BlockSpec((P, D), lambda i: (0, 0)),
            pl.BlockSpec((P, D), lambda i: (0, 0)),
            pl.BlockSpec((1, P), lambda i: (0, 0)),
        ],
        out_specs=pl.BlockSpec((BLOCK_B, D), lambda i: (i, 0)),
        out_shape=jax.ShapeDtypeStruct((B, D), jnp.float32),
        compiler_params=pltpu.CompilerParams(
            dimension_semantics=("parallel",)),
    )(x, cn, centroid_emb, bias)


# speedup vs baseline: 2.0256x; 1.0396x over previous
import jax
import jax.numpy as jnp
from jax.experimental import pallas as pl
from jax.experimental.pallas import tpu as pltpu

B, P, D = 4096, 8192, 64
BLOCK_B = 256


def _prep_kernel(c_ref, mask_ref, cn_ref, cext_ref):
    c = c_ref[...]
    m = mask_ref[...]
    cn = c / (jnp.sqrt(jnp.sum(c * c, axis=-1, keepdims=True)) + 1e-12)
    cn_ref[...] = cn.astype(jnp.bfloat16)
    mc = m.reshape(P, 1)
    cext = jnp.concatenate(
        [c * mc, mc, jnp.zeros((P, 2 * D - D - 1), jnp.float32)], axis=1)
    cext_ref[...] = cext.astype(jnp.bfloat16)


def _centroid_kernel(x_ref, cn_ref, cext_ref, out_ref):
    x = x_ref[...]
    cn = cn_ref[...]
    cext = cext_ref[...]

    xn = x / (jnp.sqrt(jnp.sum(x * x, axis=-1, keepdims=True)) + 1e-12)

    sim = jax.lax.dot_general(
        xn.astype(jnp.bfloat16), cn, (((1,), (1,)), ((), ())),
        preferred_element_type=jnp.float32)
    e = jnp.exp(sim).astype(jnp.bfloat16)

    ctx = jax.lax.dot_general(
        e, cext, (((1,), (0,)), ((), ())),
        preferred_element_type=jnp.float32)
    out_ref[...] = ctx[:, :D] / ctx[:, D:D + 1]


@jax.jit
def kernel(x, centroid_emb, active_mask):
    maskf = active_mask.astype(jnp.float32).reshape(1, P)
    cn, cext = pl.pallas_call(
        _prep_kernel,
        out_shape=[
            jax.ShapeDtypeStruct((P, D), jnp.bfloat16),
            jax.ShapeDtypeStruct((P, 2 * D), jnp.bfloat16),
        ],
    )(centroid_emb, maskf)
    return pl.pallas_call(
        _centroid_kernel,
        grid=(B // BLOCK_B,),
        in_specs=[
            pl.BlockSpec((BLOCK_B, D), lambda i: (i, 0)),
            pl.BlockSpec((P, D), lambda i: (0, 0)),
            pl.BlockSpec((P, 2 * D), lambda i: (0, 0)),
        ],
        out_specs=pl.BlockSpec((BLOCK_B, D), lambda i: (i, 0)),
        out_shape=jax.ShapeDtypeStruct((B, D), jnp.float32),
        compiler_params=pltpu.CompilerParams(
            dimension_semantics=("parallel",)),
    )(x, cn, cext)


# speedup vs baseline: 2.0618x; 1.0179x over previous
import jax
import jax.numpy as jnp
from jax.experimental import pallas as pl
from jax.experimental.pallas import tpu as pltpu

B, P, D = 4096, 8192, 64
BLOCK_B = 512


def _prep_kernel(c_ref, mask_ref, cn_ref, cext_ref):
    c = c_ref[...]
    m = mask_ref[...]
    cn = c / (jnp.sqrt(jnp.sum(c * c, axis=-1, keepdims=True)) + 1e-12)
    cn_ref[...] = cn.astype(jnp.bfloat16)
    mc = m.reshape(P, 1)
    cext = jnp.concatenate(
        [c * mc, mc, jnp.zeros((P, 2 * D - D - 1), jnp.float32)], axis=1)
    cext_ref[...] = cext.astype(jnp.bfloat16)


def _centroid_kernel(x_ref, cn_ref, cext_ref, out_ref):
    x = x_ref[...]
    cn = cn_ref[...]
    cext = cext_ref[...]

    xn = x / (jnp.sqrt(jnp.sum(x * x, axis=-1, keepdims=True)) + 1e-12)

    sim = jax.lax.dot_general(
        xn.astype(jnp.bfloat16), cn, (((1,), (1,)), ((), ())),
        preferred_element_type=jnp.float32)
    e = jnp.exp(sim).astype(jnp.bfloat16)

    ctx = jax.lax.dot_general(
        e, cext, (((1,), (0,)), ((), ())),
        preferred_element_type=jnp.float32)
    out_ref[...] = ctx[:, :D] / ctx[:, D:D + 1]


@jax.jit
def kernel(x, centroid_emb, active_mask):
    maskf = active_mask.astype(jnp.float32).reshape(1, P)
    cn, cext = pl.pallas_call(
        _prep_kernel,
        out_shape=[
            jax.ShapeDtypeStruct((P, D), jnp.bfloat16),
            jax.ShapeDtypeStruct((P, 2 * D), jnp.bfloat16),
        ],
    )(centroid_emb, maskf)
    return pl.pallas_call(
        _centroid_kernel,
        grid=(B // BLOCK_B,),
        in_specs=[
            pl.BlockSpec((BLOCK_B, D), lambda i: (i, 0)),
            pl.BlockSpec((P, D), lambda i: (0, 0)),
            pl.BlockSpec((P, 2 * D), lambda i: (0, 0)),
        ],
        out_specs=pl.BlockSpec((BLOCK_B, D), lambda i: (i, 0)),
        out_shape=jax.ShapeDtypeStruct((B, D), jnp.float32),
        compiler_params=pltpu.CompilerParams(
            dimension_semantics=("parallel",)),
    )(x, cn, cext)


# speedup vs baseline: 2.0884x; 1.0129x over previous
import jax
import jax.numpy as jnp
from jax.experimental import pallas as pl
from jax.experimental.pallas import tpu as pltpu

B, P, D = 4096, 8192, 64
BLOCK_B = 512


def _prep_kernel(c_ref, mask_ref, cn_ref, cm_ref, bias_ref):
    c = c_ref[...]
    m = mask_ref[...]
    cn = c / (jnp.sqrt(jnp.sum(c * c, axis=-1, keepdims=True)) + 1e-12)
    cn_ref[...] = cn.astype(jnp.bfloat16)
    cm_ref[...] = (c * m.reshape(P, 1)).astype(jnp.bfloat16)
    bias_ref[...] = jnp.where(m > 0, 0.0, -1e9).astype(jnp.float32)


def _centroid_kernel(x_ref, cn_ref, cm_ref, bias_ref, out_ref):
    x = x_ref[...]
    cn = cn_ref[...]
    cm = cm_ref[...]
    bias = bias_ref[...]

    xn = x / (jnp.sqrt(jnp.sum(x * x, axis=-1, keepdims=True)) + 1e-12)

    sim = jax.lax.dot_general(
        xn.astype(jnp.bfloat16), cn, (((1,), (1,)), ((), ())),
        preferred_element_type=jnp.float32)
    e = jnp.exp(sim + bias)
    s = jnp.sum(e, axis=-1, keepdims=True)

    ctx = jax.lax.dot_general(
        e.astype(jnp.bfloat16), cm, (((1,), (0,)), ((), ())),
        preferred_element_type=jnp.float32)
    out_ref[...] = ctx / s


@jax.jit
def kernel(x, centroid_emb, active_mask):
    maskf = active_mask.astype(jnp.float32).reshape(1, P)
    cn, cm, bias = pl.pallas_call(
        _prep_kernel,
        out_shape=[
            jax.ShapeDtypeStruct((P, D), jnp.bfloat16),
            jax.ShapeDtypeStruct((P, D), jnp.bfloat16),
            jax.ShapeDtypeStruct((1, P), jnp.float32),
        ],
    )(centroid_emb, maskf)
    return pl.pallas_call(
        _centroid_kernel,
        grid=(B // BLOCK_B,),
        in_specs=[
            pl.BlockSpec((BLOCK_B, D), lambda i: (i, 0)),
            pl.BlockSpec((P, D), lambda i: (0, 0)),
            pl.BlockSpec((P, D), lambda i: (0, 0)),
            pl.BlockSpec((1, P), lambda i: (0, 0)),
        ],
        out_specs=pl.BlockSpec((BLOCK_B, D), lambda i: (i, 0)),
        out_shape=jax.ShapeDtypeStruct((B, D), jnp.float32),
        compiler_params=pltpu.CompilerParams(
            dimension_semantics=("parallel",)),
    )(x, cn, cm, bias)


# speedup vs baseline: 2.1806x; 1.0441x over previous
import jax
import jax.numpy as jnp
from jax.experimental import pallas as pl
from jax.experimental.pallas import tpu as pltpu

B, P, D = 4096, 8192, 64
BLOCK_B = 512


def _centroid_kernel(x_ref, c_ref, mask_ref, out_ref, cn_ref, cm_ref, bias_ref):
    @pl.when(pl.program_id(0) == 0)
    def _prep():
        c = c_ref[...]
        m = mask_ref[...]
        cn = c / (jnp.sqrt(jnp.sum(c * c, axis=-1, keepdims=True)) + 1e-12)
        cn_ref[...] = cn.astype(jnp.bfloat16)
        cm_ref[...] = (c * m.reshape(P, 1)).astype(jnp.bfloat16)
        bias_ref[...] = jnp.where(m > 0, 0.0, -1e9).astype(jnp.float32)

    x = x_ref[...]
    xn = x / (jnp.sqrt(jnp.sum(x * x, axis=-1, keepdims=True)) + 1e-12)

    sim = jax.lax.dot_general(
        xn.astype(jnp.bfloat16), cn_ref[...], (((1,), (1,)), ((), ())),
        preferred_element_type=jnp.float32)
    e = jnp.exp(sim + bias_ref[...])
    s = jnp.sum(e, axis=-1, keepdims=True)

    ctx = jax.lax.dot_general(
        e.astype(jnp.bfloat16), cm_ref[...], (((1,), (0,)), ((), ())),
        preferred_element_type=jnp.float32)
    out_ref[...] = ctx / s


@jax.jit
def kernel(x, centroid_emb, active_mask):
    maskf = active_mask.astype(jnp.float32).reshape(1, P)
    return pl.pallas_call(
        _centroid_kernel,
        grid=(B // BLOCK_B,),
        in_specs=[
            pl.BlockSpec((BLOCK_B, D), lambda i: (i, 0)),
            pl.BlockSpec((P, D), lambda i: (0, 0)),
            pl.BlockSpec((1, P), lambda i: (0, 0)),
        ],
        out_specs=pl.BlockSpec((BLOCK_B, D), lambda i: (i, 0)),
        out_shape=jax.ShapeDtypeStruct((B, D), jnp.float32),
        scratch_shapes=[
            pltpu.VMEM((P, D), jnp.bfloat16),
            pltpu.VMEM((P, D), jnp.bfloat16),
            pltpu.VMEM((1, P), jnp.float32),
        ],
    )(x, centroid_emb, maskf)
